# baseline (speedup 1.0000x reference)
import functools

import jax
import jax.numpy as jnp
from jax import lax
from jax.experimental import pallas as pl
from jax.experimental.pallas import tpu as pltpu
from jax.experimental.pallas import tpu_sc as plsc

N_NODES = 10000
NP = 10240
N_EDGES = 320000
HID = 64
NUM_LAYERS = 7
EPS = 1e-07
C = 128
NW = 32
CHUNKS = N_EDGES // C
FULL = CHUNKS // NW
REM = CHUNKS - FULL * NW
RPT = NP // 16


def _edge_pass_body(z_hbm, e_hbm, src_hbm, dst_hbm, t_hbm, out_hbm,
                    src_v, dst_v, zrows, erows, orows, tv_v,
                    acc, gsem, esem):
    cid = lax.axis_index("c")
    sid = lax.axis_index("s")
    wid = sid * 2 + cid

    def zrow(r, carry):
        for k in range(8):
            orows[r, pl.ds(k * 16, 16)] = jnp.zeros((16,), jnp.float32)
        return carry
    lax.fori_loop(0, C, zrow, 0)
    base_r = sid * RPT
    for j in range(RPT // C):
        pltpu.sync_copy(orows, acc.at[pl.ds(base_r + j * C, C)])
    pltpu.sync_copy(t_hbm, tv_v)
    plsc.subcore_barrier()

    tv = tv_v[...]
    nchunks = jnp.where(wid < REM, FULL + 1, FULL)

    def chunk(i, carry):
        base = (i * NW + wid) * C
        pltpu.sync_copy(src_hbm.at[pl.ds(base, C)], src_v)
        pltpu.sync_copy(dst_hbm.at[pl.ds(base, C)], dst_v)
        gcopy = pltpu.async_copy(z_hbm.at[src_v], zrows, gsem)
        ecopy = pltpu.async_copy(e_hbm.at[pl.ds(base, C)], erows, esem)
        gcopy.wait()
        ecopy.wait()

        def row(r, rc):
            for k in range(4):
                sl = pl.ds(k * 16, 16)
                m = jnp.maximum(zrows[r, sl] + erows[r, sl], 0.0) + EPS
                ex = jnp.exp(m * tv)
                orows[r, sl] = ex
                orows[r, pl.ds(HID + k * 16, 16)] = ex * m
            return rc
        lax.fori_loop(0, C, row, 0)
        pltpu.sync_copy(orows, acc.at[dst_v], add=True)
        return carry

    lax.fori_loop(0, nchunks, chunk, 0)

    plsc.subcore_barrier()
    pltpu.sync_copy(acc.at[pl.ds(base_r, RPT)],
                    out_hbm.at[cid, pl.ds(base_r, RPT)])


@functools.cache
def _build_edge_pass():
    return pl.kernel(
        _edge_pass_body,
        out_type=jax.ShapeDtypeStruct((2, NP, 2 * HID), jnp.float32),
        mesh=plsc.VectorSubcoreMesh(core_axis_name="c", subcore_axis_name="s"),
        compiler_params=pltpu.CompilerParams(use_tc_tiling_on_sc=False),
        scratch_types=[
        pltpu.VMEM((C,), jnp.int32),
        pltpu.VMEM((C,), jnp.int32),
        pltpu.VMEM((C, HID), jnp.float32),
        pltpu.VMEM((C, HID), jnp.float32),
        pltpu.VMEM((C, 2 * HID), jnp.float32),
        pltpu.VMEM((16,), jnp.float32),
        pltpu.VMEM_SHARED((NP, 2 * HID), jnp.float32),
        pltpu.SemaphoreType.DMA,
        pltpu.SemaphoreType.DMA,
        ],
    )


def _ln(h, g, b, eps=1e-05):
    mu = jnp.mean(h, axis=-1, keepdims=True)
    var = jnp.mean((h - mu) ** 2, axis=-1, keepdims=True)
    return (h - mu) * lax.rsqrt(var + eps) * g + b


def _matmul_bias_body(x_ref, w_ref, b_ref, o_ref):
    o_ref[...] = (jnp.dot(x_ref[...], w_ref[...],
                          preferred_element_type=jnp.float32) + b_ref[...])


def _matmul_bias(x, w, b, block_rows):
    rows, k = x.shape
    _, out = w.shape
    return pl.pallas_call(
        _matmul_bias_body,
        grid=(rows // block_rows,),
        in_specs=[
            pl.BlockSpec((block_rows, k), lambda i: (i, 0)),
            pl.BlockSpec((k, out), lambda i: (0, 0)),
            pl.BlockSpec((1, out), lambda i: (0, 0)),
        ],
        out_specs=pl.BlockSpec((block_rows, out), lambda i: (i, 0)),
        out_shape=jax.ShapeDtypeStruct((rows, out), jnp.float32),
    )(x, w, b.reshape(1, -1))


def _mlp_body(sums_ref, zin_ref, hres_ref, w1_ref, b1_ref, lng_ref, lnb_ref,
              w2_ref, b2_ref, g2_ref, bb2_ref, hout_ref, zout_ref):
    s = sums_ref[0] + sums_ref[1]
    aggr = s[:, HID:2 * HID] / (s[:, 0:HID] + 1e-16)
    u = aggr + zin_ref[...]
    h1 = (jnp.dot(u, w1_ref[...], preferred_element_type=jnp.float32)
          + b1_ref[...])
    h1 = jnp.maximum(_ln(h1, lng_ref[...], lnb_ref[...]), 0.0)
    mlp = (jnp.dot(h1, w2_ref[...], preferred_element_type=jnp.float32)
           + b2_ref[...])
    hn = hres_ref[...] + mlp
    hout_ref[...] = hn
    zout_ref[...] = jnp.maximum(_ln(hn, g2_ref[...], bb2_ref[...]), 0.0)


def _mlp_call(sums, zin, hres, w1, b1, lng, lnb, w2, b2, g2, bb2):
    br = 1000
    grid = N_NODES // br
    h2 = 2 * HID
    vec = lambda a: a.reshape(1, -1)
    return pl.pallas_call(
        _mlp_body,
        grid=(grid,),
        in_specs=[
            pl.BlockSpec((2, br, h2), lambda i: (0, i, 0)),
            pl.BlockSpec((br, HID), lambda i: (i, 0)),
            pl.BlockSpec((br, HID), lambda i: (i, 0)),
            pl.BlockSpec((HID, h2), lambda i: (0, 0)),
            pl.BlockSpec((1, h2), lambda i: (0, 0)),
            pl.BlockSpec((1, h2), lambda i: (0, 0)),
            pl.BlockSpec((1, h2), lambda i: (0, 0)),
            pl.BlockSpec((h2, HID), lambda i: (0, 0)),
            pl.BlockSpec((1, HID), lambda i: (0, 0)),
            pl.BlockSpec((1, HID), lambda i: (0, 0)),
            pl.BlockSpec((1, HID), lambda i: (0, 0)),
        ],
        out_specs=[
            pl.BlockSpec((br, HID), lambda i: (i, 0)),
            pl.BlockSpec((br, HID), lambda i: (i, 0)),
        ],
        out_shape=[
            jax.ShapeDtypeStruct((N_NODES, HID), jnp.float32),
            jax.ShapeDtypeStruct((N_NODES, HID), jnp.float32),
        ],
    )(sums, zin, hres, w1, vec(b1), vec(lng), vec(lnb), w2, vec(b2),
      vec(g2), vec(bb2))


def kernel(x, edge_index, edge_attr, node_W, node_b, edge_W, edge_b,
           mlp_W1, mlp_b1, mlp_lng, mlp_lnb, mlp_W2, mlp_b2, t,
           ln_g, ln_b, lin_W, lin_b):
    src = edge_index[0]
    dst = edge_index[1]
    z = _matmul_bias(x, node_W, node_b, 1000)
    e = _matmul_bias(edge_attr, edge_W, edge_b, 8000)

    hres = jnp.zeros((N_NODES, HID), jnp.float32)
    for i in range(NUM_LAYERS):
        t16 = jnp.full((16,), t[i], jnp.float32)
        sums = _build_edge_pass()(z, e, src, dst, t16)
        nxt = (i + 1) % NUM_LAYERS
        h, z = _mlp_call(sums, z, hres, mlp_W1[i], mlp_b1[i], mlp_lng[i],
                         mlp_lnb[i], mlp_W2[i], mlp_b2[i],
                         ln_g[nxt], ln_b[nxt])
        hres = h
    return _matmul_bias(z, lin_W, lin_b, 1000)


# speedup vs baseline: 2.3065x; 2.3065x over previous
import functools

import jax
import jax.numpy as jnp
from jax import lax
from jax.experimental import pallas as pl
from jax.experimental.pallas import tpu as pltpu
from jax.experimental.pallas import tpu_sc as plsc

N_NODES = 10000
NP = 10112
N_EDGES = 320000
HID = 64
NUM_LAYERS = 7
EPS = 1e-07
C = 64
NW = 32
CHUNKS = N_EDGES // C
FULL = CHUNKS // NW
REM = CHUNKS - FULL * NW
PAIRS = FULL // 2
RPT = NP // 16


def _edge_pass_body(z_hbm, e_hbm, src_hbm, dst_hbm, t_hbm, out_hbm,
                    src0, dst0, src1, dst1, z0, z1, e0, e1, tv_v, acc,
                    gsem0, gsem1, esem0, esem1, ssem0, ssem1):
    cid = lax.axis_index("c")
    sid = lax.axis_index("s")
    wid = sid * 2 + cid

    def zfill(r, carry):
        for k in range(8):
            z0[r, pl.ds(k * 16, 16)] = jnp.zeros((16,), jnp.float32)
        return carry
    lax.fori_loop(0, C, zfill, 0)
    base_r = sid * RPT
    nfull = RPT // C
    for j in range(nfull):
        pltpu.sync_copy(z0, acc.at[pl.ds(base_r + j * C, C)])
    tail = RPT - nfull * C
    if tail:
        pltpu.sync_copy(z0.at[pl.ds(0, tail)],
                        acc.at[pl.ds(base_r + nfull * C, tail)])
    pltpu.sync_copy(t_hbm, tv_v)
    plsc.subcore_barrier()

    tv = tv_v[...]

    def compute(zbuf, ebuf):
        def row(r, rc):
            for k in range(4):
                sl = pl.ds(k * 16, 16)
                m = jnp.maximum(zbuf[r, sl] + ebuf[r, sl], 0.0) + EPS
                ex = jnp.exp(m * tv)
                zbuf[r, sl] = ex
                zbuf[r, pl.ds(HID + k * 16, 16)] = ex * m
            return rc
        lax.fori_loop(0, C, row, 0)

    @pl.when(wid < REM)
    def _rem():
        base = (FULL * NW + wid) * C
        pltpu.sync_copy(src_hbm.at[pl.ds(base, C)], src0)
        pltpu.sync_copy(dst_hbm.at[pl.ds(base, C)], dst0)
        pltpu.async_copy(z_hbm.at[src0], z0, gsem0).wait()
        pltpu.sync_copy(e_hbm.at[pl.ds(base, C)], e0)
        compute(z0, e0)
        pltpu.sync_copy(z0, acc.at[dst0], add=True)

    def issue(i, sbuf, dbuf, zbuf, ebuf, gsem, esem):
        base = (i * NW + wid) * C
        pltpu.sync_copy(src_hbm.at[pl.ds(base, C)], sbuf)
        pltpu.sync_copy(dst_hbm.at[pl.ds(base, C)], dbuf)
        pltpu.async_copy(z_hbm.at[sbuf], zbuf, gsem)
        pltpu.async_copy(e_hbm.at[pl.ds(base, C)], ebuf, esem)

    issue(0, src0, dst0, z0, e0, gsem0, esem0)
    issue(1, src1, dst1, z1, e1, gsem1, esem1)

    def pair(p, carry):
        i0 = 2 * p
        i1 = 2 * p + 1
        @pl.when(p > 0)
        def _():
            pltpu.make_async_copy(z1, acc.at[dst1], ssem1).wait()
            issue(i1, src1, dst1, z1, e1, gsem1, esem1)
        pltpu.make_async_copy(z_hbm.at[src0], z0, gsem0).wait()
        pltpu.make_async_copy(e_hbm.at[pl.ds(0, C)], e0, esem0).wait()
        compute(z0, e0)
        pltpu.async_copy(z0, acc.at[dst0], ssem0, add=True)
        pltpu.make_async_copy(z_hbm.at[src1], z1, gsem1).wait()
        pltpu.make_async_copy(e_hbm.at[pl.ds(0, C)], e1, esem1).wait()
        @pl.when(p < PAIRS - 1)
        def _():
            pltpu.make_async_copy(z0, acc.at[dst0], ssem0).wait()
            issue(i0 + 2, src0, dst0, z0, e0, gsem0, esem0)
        compute(z1, e1)
        pltpu.async_copy(z1, acc.at[dst1], ssem1, add=True)
        return carry

    lax.fori_loop(0, PAIRS, pair, 0)
    pltpu.make_async_copy(z0, acc.at[dst0], ssem0).wait()
    pltpu.make_async_copy(z1, acc.at[dst1], ssem1).wait()

    plsc.subcore_barrier()
    pltpu.sync_copy(acc.at[pl.ds(base_r, RPT)],
                    out_hbm.at[cid, pl.ds(base_r, RPT)])


@functools.cache
def _build_edge_pass():
    return pl.kernel(
        _edge_pass_body,
        out_type=jax.ShapeDtypeStruct((2, NP, 2 * HID), jnp.float32),
        mesh=plsc.VectorSubcoreMesh(core_axis_name="c", subcore_axis_name="s"),
        scratch_types=[
            pltpu.VMEM((C,), jnp.int32),
            pltpu.VMEM((C,), jnp.int32),
            pltpu.VMEM((C,), jnp.int32),
            pltpu.VMEM((C,), jnp.int32),
            pltpu.VMEM((C, 2 * HID), jnp.float32),
            pltpu.VMEM((C, 2 * HID), jnp.float32),
            pltpu.VMEM((C, HID), jnp.float32),
            pltpu.VMEM((C, HID), jnp.float32),
            pltpu.VMEM((16,), jnp.float32),
            pltpu.VMEM_SHARED((NP, 2 * HID), jnp.float32),
            pltpu.SemaphoreType.DMA,
            pltpu.SemaphoreType.DMA,
            pltpu.SemaphoreType.DMA,
            pltpu.SemaphoreType.DMA,
            pltpu.SemaphoreType.DMA,
            pltpu.SemaphoreType.DMA,
        ],
    )


def _ln(h, g, b, eps=1e-05):
    mu = jnp.mean(h, axis=-1, keepdims=True)
    var = jnp.mean((h - mu) ** 2, axis=-1, keepdims=True)
    return (h - mu) * lax.rsqrt(var + eps) * g + b


def _matmul_bias_body(x_ref, w_ref, b_ref, o_ref):
    o_ref[...] = (jnp.dot(x_ref[...], w_ref[...],
                          preferred_element_type=jnp.float32) + b_ref[...])


def _matmul_bias(x, w, b, block_rows):
    rows, k = x.shape
    _, out = w.shape
    return pl.pallas_call(
        _matmul_bias_body,
        grid=(rows // block_rows,),
        in_specs=[
            pl.BlockSpec((block_rows, k), lambda i: (i, 0)),
            pl.BlockSpec((k, out), lambda i: (0, 0)),
            pl.BlockSpec((1, out), lambda i: (0, 0)),
        ],
        out_specs=pl.BlockSpec((block_rows, out), lambda i: (i, 0)),
        out_shape=jax.ShapeDtypeStruct((rows, out), jnp.float32),
    )(x, w, b.reshape(1, -1))


def _mlp_body(sums_ref, zin_ref, hres_ref, w1_ref, b1_ref, lng_ref, lnb_ref,
              w2_ref, b2_ref, g2_ref, bb2_ref, hout_ref, zout_ref):
    s = sums_ref[0] + sums_ref[1]
    aggr = s[:, HID:2 * HID] / (s[:, 0:HID] + 1e-16)
    u = aggr + zin_ref[:, 0:HID]
    h1 = (jnp.dot(u, w1_ref[...], preferred_element_type=jnp.float32)
          + b1_ref[...])
    h1 = jnp.maximum(_ln(h1, lng_ref[...], lnb_ref[...]), 0.0)
    mlp = (jnp.dot(h1, w2_ref[...], preferred_element_type=jnp.float32)
           + b2_ref[...])
    hn = hres_ref[...] + mlp
    hout_ref[...] = hn
    zn = jnp.maximum(_ln(hn, g2_ref[...], bb2_ref[...]), 0.0)
    zout_ref[...] = jnp.concatenate([zn, jnp.zeros_like(zn)], axis=1)


def _mlp_call(sums, zin, hres, w1, b1, lng, lnb, w2, b2, g2, bb2):
    br = 1000
    grid = N_NODES // br
    h2 = 2 * HID
    vec = lambda a: a.reshape(1, -1)
    return pl.pallas_call(
        _mlp_body,
        grid=(grid,),
        in_specs=[
            pl.BlockSpec((2, br, h2), lambda i: (0, i, 0)),
            pl.BlockSpec((br, h2), lambda i: (i, 0)),
            pl.BlockSpec((br, HID), lambda i: (i, 0)),
            pl.BlockSpec((HID, h2), lambda i: (0, 0)),
            pl.BlockSpec((1, h2), lambda i: (0, 0)),
            pl.BlockSpec((1, h2), lambda i: (0, 0)),
            pl.BlockSpec((1, h2), lambda i: (0, 0)),
            pl.BlockSpec((h2, HID), lambda i: (0, 0)),
            pl.BlockSpec((1, HID), lambda i: (0, 0)),
            pl.BlockSpec((1, HID), lambda i: (0, 0)),
            pl.BlockSpec((1, HID), lambda i: (0, 0)),
        ],
        out_specs=[
            pl.BlockSpec((br, HID), lambda i: (i, 0)),
            pl.BlockSpec((br, h2), lambda i: (i, 0)),
        ],
        out_shape=[
            jax.ShapeDtypeStruct((N_NODES, HID), jnp.float32),
            jax.ShapeDtypeStruct((N_NODES, h2), jnp.float32),
        ],
    )(sums, zin, hres, w1, vec(b1), vec(lng), vec(lnb), w2, vec(b2),
      vec(g2), vec(bb2))


def kernel(x, edge_index, edge_attr, node_W, node_b, edge_W, edge_b,
           mlp_W1, mlp_b1, mlp_lng, mlp_lnb, mlp_W2, mlp_b2, t,
           ln_g, ln_b, lin_W, lin_b):
    src = edge_index[0]
    dst = edge_index[1]
    nw_pad = jnp.pad(node_W, ((0, 0), (0, HID)))
    nb_pad = jnp.pad(node_b, (0, HID))
    z = _matmul_bias(x, nw_pad, nb_pad, 1000)
    e = _matmul_bias(edge_attr, edge_W, edge_b, 8000)

    hres = jnp.zeros((N_NODES, HID), jnp.float32)
    for i in range(NUM_LAYERS):
        t16 = jnp.full((16,), t[i], jnp.float32)
        sums = _build_edge_pass()(z, e, src, dst, t16)
        nxt = (i + 1) % NUM_LAYERS
        h, z = _mlp_call(sums, z, hres, mlp_W1[i], mlp_b1[i], mlp_lng[i],
                         mlp_lnb[i], mlp_W2[i], mlp_b2[i],
                         ln_g[nxt], ln_b[nxt])
        hres = h
    lw_pad = jnp.pad(lin_W, ((0, HID), (0, 0)))
    return _matmul_bias(z, lw_pad, lin_b, 1000)


# speedup vs baseline: 3.3588x; 1.4562x over previous
import functools

import jax
import jax.numpy as jnp
from jax import lax
from jax.experimental import pallas as pl
from jax.experimental.pallas import tpu as pltpu
from jax.experimental.pallas import tpu_sc as plsc

N_NODES = 10000
NP = 10112
N_EDGES = 320000
HID = 64
NUM_LAYERS = 7
EPS = 1e-07
C = 64
NW = 32
CHUNKS = N_EDGES // C
FULL = CHUNKS // NW
REM = CHUNKS - FULL * NW
RPT = NP // 16


def _edge_pass_body(z_hbm, e_hbm, pk_hbm, t_hbm, out_hbm,
                    pbuf, src0, dst0, src1, dst1, z0, z1, e0, e1, tv_v, acc,
                    gsem0, gsem1, esem0, esem1, ssem0, ssem1):
    cid = lax.axis_index("c")
    sid = lax.axis_index("s")
    wid = sid * 2 + cid

    def zfill(r, carry):
        for k in range(8):
            z0[r, pl.ds(k * 16, 16)] = jnp.zeros((16,), jnp.float32)
        return carry
    lax.fori_loop(0, C, zfill, 0)
    base_r = sid * RPT
    nfull = RPT // C
    for j in range(nfull):
        pltpu.sync_copy(z0, acc.at[pl.ds(base_r + j * C, C)])
    tail = RPT - nfull * C
    if tail:
        pltpu.sync_copy(z0.at[pl.ds(0, tail)],
                        acc.at[pl.ds(base_r + nfull * C, tail)])
    pltpu.sync_copy(t_hbm, tv_v)
    plsc.subcore_barrier()

    tv = tv_v[...]

    def unpack(r, sbuf, dbuf):
        for k in range(4):
            sl = pl.ds(k * 16, 16)
            v = pbuf[pl.ds(r * C + k * 16, 16)]
            sbuf[sl] = v & 0xFFFF
            dbuf[sl] = lax.shift_right_logical(v, 16)

    def compute(zbuf, ebuf):
        def row(r, rc):
            for k in range(4):
                sl = pl.ds(k * 16, 16)
                m = jnp.maximum(zbuf[r, sl] + ebuf[r, sl], 0.0) + EPS
                ex = jnp.exp(m * tv)
                zbuf[r, sl] = ex
                zbuf[r, pl.ds(HID + k * 16, 16)] = ex * m
            return rc
        lax.fori_loop(0, C, row, 0)

    @pl.when(wid < REM)
    def _rem():
        pltpu.sync_copy(pk_hbm.at[pl.ds((FULL * NW + wid) * C, C)],
                        pbuf.at[pl.ds(0, C)])
        unpack(0, src0, dst0)
        pltpu.async_copy(z_hbm.at[src0], z0, gsem0).wait()
        base = (FULL * NW + wid) * C
        pltpu.sync_copy(e_hbm.at[pl.ds(base, C)], e0)
        compute(z0, e0)
        pltpu.sync_copy(z0, acc.at[dst0], add=True)

    HC = FULL // 2
    HPAIRS = HC // 2

    for half in range(2):
        chunk0 = wid * FULL + half * HC
        pltpu.sync_copy(pk_hbm.at[pl.ds(chunk0 * C, HC * C)], pbuf)

        def issue(r, sbuf, dbuf, zbuf, ebuf, gsem, esem):
            unpack(r, sbuf, dbuf)
            pltpu.async_copy(z_hbm.at[sbuf], zbuf, gsem)
            pltpu.async_copy(e_hbm.at[pl.ds((chunk0 + r) * C, C)], ebuf, esem)

        issue(0, src0, dst0, z0, e0, gsem0, esem0)
        issue(1, src1, dst1, z1, e1, gsem1, esem1)

        def pair(p, carry):
            @pl.when(p > 0)
            def _():
                pltpu.make_async_copy(z1, acc.at[dst1], ssem1).wait()
                issue(2 * p + 1, src1, dst1, z1, e1, gsem1, esem1)
            pltpu.make_async_copy(z_hbm.at[src0], z0, gsem0).wait()
            pltpu.make_async_copy(e_hbm.at[pl.ds(0, C)], e0, esem0).wait()
            compute(z0, e0)
            pltpu.async_copy(z0, acc.at[dst0], ssem0, add=True)
            pltpu.make_async_copy(z_hbm.at[src1], z1, gsem1).wait()
            pltpu.make_async_copy(e_hbm.at[pl.ds(0, C)], e1, esem1).wait()
            @pl.when(p < HPAIRS - 1)
            def _():
                pltpu.make_async_copy(z0, acc.at[dst0], ssem0).wait()
                issue(2 * p + 2, src0, dst0, z0, e0, gsem0, esem0)
            compute(z1, e1)
            pltpu.async_copy(z1, acc.at[dst1], ssem1, add=True)
            return carry

        lax.fori_loop(0, HPAIRS, pair, 0)
        pltpu.make_async_copy(z0, acc.at[dst0], ssem0).wait()
        pltpu.make_async_copy(z1, acc.at[dst1], ssem1).wait()

    plsc.subcore_barrier()
    pltpu.sync_copy(acc.at[pl.ds(base_r, RPT)],
                    out_hbm.at[cid, pl.ds(base_r, RPT)])


@functools.cache
def _build_edge_pass():
    return pl.kernel(
        _edge_pass_body,
        out_type=jax.ShapeDtypeStruct((2, NP, 2 * HID), jnp.float32),
        mesh=plsc.VectorSubcoreMesh(core_axis_name="c", subcore_axis_name="s"),
        scratch_types=[
            pltpu.VMEM((FULL // 2 * C,), jnp.int32),
            pltpu.VMEM((C,), jnp.int32),
            pltpu.VMEM((C,), jnp.int32),
            pltpu.VMEM((C,), jnp.int32),
            pltpu.VMEM((C,), jnp.int32),
            pltpu.VMEM((C, 2 * HID), jnp.float32),
            pltpu.VMEM((C, 2 * HID), jnp.float32),
            pltpu.VMEM((C, HID), jnp.float32),
            pltpu.VMEM((C, HID), jnp.float32),
            pltpu.VMEM((16,), jnp.float32),
            pltpu.VMEM_SHARED((NP, 2 * HID), jnp.float32),
            pltpu.SemaphoreType.DMA,
            pltpu.SemaphoreType.DMA,
            pltpu.SemaphoreType.DMA,
            pltpu.SemaphoreType.DMA,
            pltpu.SemaphoreType.DMA,
            pltpu.SemaphoreType.DMA,
        ],
    )


def _ln(h, g, b, eps=1e-05):
    mu = jnp.mean(h, axis=-1, keepdims=True)
    var = jnp.mean((h - mu) ** 2, axis=-1, keepdims=True)
    return (h - mu) * lax.rsqrt(var + eps) * g + b


def _matmul_bias_body(x_ref, w_ref, b_ref, o_ref):
    o_ref[...] = (jnp.dot(x_ref[...], w_ref[...],
                          preferred_element_type=jnp.float32) + b_ref[...])


def _matmul_bias(x, w, b, block_rows):
    rows, k = x.shape
    _, out = w.shape
    return pl.pallas_call(
        _matmul_bias_body,
        grid=(rows // block_rows,),
        in_specs=[
            pl.BlockSpec((block_rows, k), lambda i: (i, 0)),
            pl.BlockSpec((k, out), lambda i: (0, 0)),
            pl.BlockSpec((1, out), lambda i: (0, 0)),
        ],
        out_specs=pl.BlockSpec((block_rows, out), lambda i: (i, 0)),
        out_shape=jax.ShapeDtypeStruct((rows, out), jnp.float32),
    )(x, w, b.reshape(1, -1))


def _mlp_body(sums_ref, zin_ref, hres_ref, w1_ref, b1_ref, lng_ref, lnb_ref,
              w2_ref, b2_ref, g2_ref, bb2_ref, hout_ref, zout_ref):
    s = sums_ref[0] + sums_ref[1]
    aggr = s[:, HID:2 * HID] / (s[:, 0:HID] + 1e-16)
    u = aggr + zin_ref[:, 0:HID]
    h1 = (jnp.dot(u, w1_ref[...], preferred_element_type=jnp.float32)
          + b1_ref[...])
    h1 = jnp.maximum(_ln(h1, lng_ref[...], lnb_ref[...]), 0.0)
    mlp = (jnp.dot(h1, w2_ref[...], preferred_element_type=jnp.float32)
           + b2_ref[...])
    hn = hres_ref[...] + mlp
    hout_ref[...] = hn
    zn = jnp.maximum(_ln(hn, g2_ref[...], bb2_ref[...]), 0.0)
    zout_ref[...] = jnp.concatenate([zn, jnp.zeros_like(zn)], axis=1)


def _mlp_call(sums, zin, hres, w1, b1, lng, lnb, w2, b2, g2, bb2):
    br = 1000
    grid = N_NODES // br
    h2 = 2 * HID
    vec = lambda a: a.reshape(1, -1)
    return pl.pallas_call(
        _mlp_body,
        grid=(grid,),
        in_specs=[
            pl.BlockSpec((2, br, h2), lambda i: (0, i, 0)),
            pl.BlockSpec((br, h2), lambda i: (i, 0)),
            pl.BlockSpec((br, HID), lambda i: (i, 0)),
            pl.BlockSpec((HID, h2), lambda i: (0, 0)),
            pl.BlockSpec((1, h2), lambda i: (0, 0)),
            pl.BlockSpec((1, h2), lambda i: (0, 0)),
            pl.BlockSpec((1, h2), lambda i: (0, 0)),
            pl.BlockSpec((h2, HID), lambda i: (0, 0)),
            pl.BlockSpec((1, HID), lambda i: (0, 0)),
            pl.BlockSpec((1, HID), lambda i: (0, 0)),
            pl.BlockSpec((1, HID), lambda i: (0, 0)),
        ],
        out_specs=[
            pl.BlockSpec((br, HID), lambda i: (i, 0)),
            pl.BlockSpec((br, h2), lambda i: (i, 0)),
        ],
        out_shape=[
            jax.ShapeDtypeStruct((N_NODES, HID), jnp.float32),
            jax.ShapeDtypeStruct((N_NODES, h2), jnp.float32),
        ],
    )(sums, zin, hres, w1, vec(b1), vec(lng), vec(lnb), w2, vec(b2),
      vec(g2), vec(bb2))


def kernel(x, edge_index, edge_attr, node_W, node_b, edge_W, edge_b,
           mlp_W1, mlp_b1, mlp_lng, mlp_lnb, mlp_W2, mlp_b2, t,
           ln_g, ln_b, lin_W, lin_b):
    src = edge_index[0]
    dst = edge_index[1]
    packed = src | (dst << 16)
    nw_pad = jnp.pad(node_W, ((0, 0), (0, HID)))
    nb_pad = jnp.pad(node_b, (0, HID))
    z = _matmul_bias(x, nw_pad, nb_pad, 1000)
    e = _matmul_bias(edge_attr, edge_W, edge_b, 8000)

    hres = jnp.zeros((N_NODES, HID), jnp.float32)
    for i in range(NUM_LAYERS):
        t16 = jnp.full((16,), t[i], jnp.float32)
        sums = _build_edge_pass()(z, e, packed, t16)
        nxt = (i + 1) % NUM_LAYERS
        h, z = _mlp_call(sums, z, hres, mlp_W1[i], mlp_b1[i], mlp_lng[i],
                         mlp_lnb[i], mlp_W2[i], mlp_b2[i],
                         ln_g[nxt], ln_b[nxt])
        hres = h
    lw_pad = jnp.pad(lin_W, ((0, HID), (0, 0)))
    return _matmul_bias(z, lw_pad, lin_b, 1000)


# speedup vs baseline: 3.7684x; 1.1220x over previous
import functools

import jax
import jax.numpy as jnp
from jax import lax
from jax.experimental import pallas as pl
from jax.experimental.pallas import tpu as pltpu
from jax.experimental.pallas import tpu_sc as plsc

N_NODES = 10000
NP = 10112
N_EDGES = 320000
HID = 64
NUM_LAYERS = 7
EPS = 1e-07
C = 64
NW = 32
CHUNKS = N_EDGES // C
FULL = CHUNKS // NW
REM = CHUNKS - FULL * NW
RPT = NP // 16


def _edge_pass_body(z_hbm, e_hbm, pk_hbm, t_hbm, out_hbm,
                    pbuf, src0, dst0, src1, dst1, z0, z1, e0, e1, tv_v, acc,
                    gsem0, gsem1, esem0, esem1, ssem0, ssem1):
    cid = lax.axis_index("c")
    sid = lax.axis_index("s")
    wid = sid * 2 + cid

    def zfill(r, carry):
        for k in range(8):
            z0[r, pl.ds(k * 16, 16)] = jnp.zeros((16,), jnp.float32)
        return carry
    lax.fori_loop(0, C, zfill, 0)
    base_r = sid * RPT
    nfull = RPT // C
    for j in range(nfull):
        pltpu.sync_copy(z0, acc.at[pl.ds(base_r + j * C, C)])
    tail = RPT - nfull * C
    if tail:
        pltpu.sync_copy(z0.at[pl.ds(0, tail)],
                        acc.at[pl.ds(base_r + nfull * C, tail)])
    pltpu.sync_copy(t_hbm, tv_v)
    plsc.subcore_barrier()

    tv = tv_v[...]

    def unpack(r, sbuf, dbuf):
        for k in range(4):
            sl = pl.ds(k * 16, 16)
            v = pbuf[pl.ds(r * C + k * 16, 16)]
            sbuf[sl] = v & 0xFFFF
            dbuf[sl] = lax.shift_right_logical(v, 16)

    def compute(zbuf, ebuf):
        @plsc.parallel_loop(0, C, 1, unroll=2)
        def row(r):
            for k in range(4):
                sl = pl.ds(k * 16, 16)
                m = jnp.maximum(zbuf[r, sl] + ebuf[r, sl], 0.0) + EPS
                ex = jnp.exp(m * tv)
                zbuf[r, sl] = ex
                zbuf[r, pl.ds(HID + k * 16, 16)] = ex * m

    @pl.when(wid < REM)
    def _rem():
        pltpu.sync_copy(pk_hbm.at[pl.ds((FULL * NW + wid) * C, C)],
                        pbuf.at[pl.ds(0, C)])
        unpack(0, src0, dst0)
        pltpu.async_copy(z_hbm.at[src0], z0, gsem0).wait()
        base = (FULL * NW + wid) * C
        pltpu.sync_copy(e_hbm.at[pl.ds(base, C)], e0)
        compute(z0, e0)
        pltpu.sync_copy(z0, acc.at[dst0], add=True)

    HC = FULL // 2
    HPAIRS = HC // 2

    for half in range(2):
        chunk0 = wid * FULL + half * HC
        pltpu.sync_copy(pk_hbm.at[pl.ds(chunk0 * C, HC * C)], pbuf)

        def issue(r, sbuf, dbuf, zbuf, ebuf, gsem, esem):
            unpack(r, sbuf, dbuf)
            pltpu.async_copy(z_hbm.at[sbuf], zbuf, gsem)
            pltpu.async_copy(e_hbm.at[pl.ds((chunk0 + r) * C, C)], ebuf, esem)

        issue(0, src0, dst0, z0, e0, gsem0, esem0)
        issue(1, src1, dst1, z1, e1, gsem1, esem1)

        def pair(p, carry):
            @pl.when(p > 0)
            def _():
                pltpu.make_async_copy(z1, acc.at[dst1], ssem1).wait()
                issue(2 * p + 1, src1, dst1, z1, e1, gsem1, esem1)
            pltpu.make_async_copy(z_hbm.at[src0], z0, gsem0).wait()
            pltpu.make_async_copy(e_hbm.at[pl.ds(0, C)], e0, esem0).wait()
            compute(z0, e0)
            pltpu.async_copy(z0, acc.at[dst0], ssem0, add=True)
            pltpu.make_async_copy(z_hbm.at[src1], z1, gsem1).wait()
            pltpu.make_async_copy(e_hbm.at[pl.ds(0, C)], e1, esem1).wait()
            @pl.when(p < HPAIRS - 1)
            def _():
                pltpu.make_async_copy(z0, acc.at[dst0], ssem0).wait()
                issue(2 * p + 2, src0, dst0, z0, e0, gsem0, esem0)
            compute(z1, e1)
            pltpu.async_copy(z1, acc.at[dst1], ssem1, add=True)
            return carry

        lax.fori_loop(0, HPAIRS, pair, 0)
        pltpu.make_async_copy(z0, acc.at[dst0], ssem0).wait()
        pltpu.make_async_copy(z1, acc.at[dst1], ssem1).wait()

    plsc.subcore_barrier()
    pltpu.sync_copy(acc.at[pl.ds(base_r, RPT)],
                    out_hbm.at[cid, pl.ds(base_r, RPT)])


@functools.cache
def _build_edge_pass():
    return pl.kernel(
        _edge_pass_body,
        out_type=jax.ShapeDtypeStruct((2, NP, 2 * HID), jnp.float32),
        mesh=plsc.VectorSubcoreMesh(core_axis_name="c", subcore_axis_name="s"),
        scratch_types=[
            pltpu.VMEM((FULL // 2 * C,), jnp.int32),
            pltpu.VMEM((C,), jnp.int32),
            pltpu.VMEM((C,), jnp.int32),
            pltpu.VMEM((C,), jnp.int32),
            pltpu.VMEM((C,), jnp.int32),
            pltpu.VMEM((C, 2 * HID), jnp.float32),
            pltpu.VMEM((C, 2 * HID), jnp.float32),
            pltpu.VMEM((C, HID), jnp.float32),
            pltpu.VMEM((C, HID), jnp.float32),
            pltpu.VMEM((16,), jnp.float32),
            pltpu.VMEM_SHARED((NP, 2 * HID), jnp.float32),
            pltpu.SemaphoreType.DMA,
            pltpu.SemaphoreType.DMA,
            pltpu.SemaphoreType.DMA,
            pltpu.SemaphoreType.DMA,
            pltpu.SemaphoreType.DMA,
            pltpu.SemaphoreType.DMA,
        ],
    )


def _ln(h, g, b, eps=1e-05):
    mu = jnp.mean(h, axis=-1, keepdims=True)
    var = jnp.mean((h - mu) ** 2, axis=-1, keepdims=True)
    return (h - mu) * lax.rsqrt(var + eps) * g + b


def _matmul_bias_body(x_ref, w_ref, b_ref, o_ref):
    o_ref[...] = (jnp.dot(x_ref[...], w_ref[...],
                          preferred_element_type=jnp.float32) + b_ref[...])


def _matmul_bias(x, w, b, block_rows):
    rows, k = x.shape
    _, out = w.shape
    return pl.pallas_call(
        _matmul_bias_body,
        grid=(rows // block_rows,),
        in_specs=[
            pl.BlockSpec((block_rows, k), lambda i: (i, 0)),
            pl.BlockSpec((k, out), lambda i: (0, 0)),
            pl.BlockSpec((1, out), lambda i: (0, 0)),
        ],
        out_specs=pl.BlockSpec((block_rows, out), lambda i: (i, 0)),
        out_shape=jax.ShapeDtypeStruct((rows, out), jnp.float32),
    )(x, w, b.reshape(1, -1))


def _mlp_body(sums_ref, zin_ref, hres_ref, w1_ref, b1_ref, lng_ref, lnb_ref,
              w2_ref, b2_ref, g2_ref, bb2_ref, hout_ref, zout_ref):
    s = sums_ref[0] + sums_ref[1]
    aggr = s[:, HID:2 * HID] / (s[:, 0:HID] + 1e-16)
    u = aggr + zin_ref[:, 0:HID]
    h1 = (jnp.dot(u, w1_ref[...], preferred_element_type=jnp.float32)
          + b1_ref[...])
    h1 = jnp.maximum(_ln(h1, lng_ref[...], lnb_ref[...]), 0.0)
    mlp = (jnp.dot(h1, w2_ref[...], preferred_element_type=jnp.float32)
           + b2_ref[...])
    hn = hres_ref[...] + mlp
    hout_ref[...] = hn
    zn = jnp.maximum(_ln(hn, g2_ref[...], bb2_ref[...]), 0.0)
    zout_ref[...] = jnp.concatenate([zn, jnp.zeros_like(zn)], axis=1)


def _mlp_call(sums, zin, hres, w1, b1, lng, lnb, w2, b2, g2, bb2):
    br = 1000
    grid = N_NODES // br
    h2 = 2 * HID
    vec = lambda a: a.reshape(1, -1)
    return pl.pallas_call(
        _mlp_body,
        grid=(grid,),
        in_specs=[
            pl.BlockSpec((2, br, h2), lambda i: (0, i, 0)),
            pl.BlockSpec((br, h2), lambda i: (i, 0)),
            pl.BlockSpec((br, HID), lambda i: (i, 0)),
            pl.BlockSpec((HID, h2), lambda i: (0, 0)),
            pl.BlockSpec((1, h2), lambda i: (0, 0)),
            pl.BlockSpec((1, h2), lambda i: (0, 0)),
            pl.BlockSpec((1, h2), lambda i: (0, 0)),
            pl.BlockSpec((h2, HID), lambda i: (0, 0)),
            pl.BlockSpec((1, HID), lambda i: (0, 0)),
            pl.BlockSpec((1, HID), lambda i: (0, 0)),
            pl.BlockSpec((1, HID), lambda i: (0, 0)),
        ],
        out_specs=[
            pl.BlockSpec((br, HID), lambda i: (i, 0)),
            pl.BlockSpec((br, h2), lambda i: (i, 0)),
        ],
        out_shape=[
            jax.ShapeDtypeStruct((N_NODES, HID), jnp.float32),
            jax.ShapeDtypeStruct((N_NODES, h2), jnp.float32),
        ],
    )(sums, zin, hres, w1, vec(b1), vec(lng), vec(lnb), w2, vec(b2),
      vec(g2), vec(bb2))


def kernel(x, edge_index, edge_attr, node_W, node_b, edge_W, edge_b,
           mlp_W1, mlp_b1, mlp_lng, mlp_lnb, mlp_W2, mlp_b2, t,
           ln_g, ln_b, lin_W, lin_b):
    src = edge_index[0]
    dst = edge_index[1]
    packed = src | (dst << 16)
    nw_pad = jnp.pad(node_W, ((0, 0), (0, HID)))
    nb_pad = jnp.pad(node_b, (0, HID))
    z = _matmul_bias(x, nw_pad, nb_pad, 1000)
    e = _matmul_bias(edge_attr, edge_W, edge_b, 8000)

    hres = jnp.zeros((N_NODES, HID), jnp.float32)
    for i in range(NUM_LAYERS):
        t16 = jnp.full((16,), t[i], jnp.float32)
        sums = _build_edge_pass()(z, e, packed, t16)
        nxt = (i + 1) % NUM_LAYERS
        h, z = _mlp_call(sums, z, hres, mlp_W1[i], mlp_b1[i], mlp_lng[i],
                         mlp_lnb[i], mlp_W2[i], mlp_b2[i],
                         ln_g[nxt], ln_b[nxt])
        hres = h
    lw_pad = jnp.pad(lin_W, ((0, HID), (0, 0)))
    return _matmul_bias(z, lw_pad, lin_b, 1000)
